# Initial kernel scaffold; baseline (speedup 1.0000x reference)
import jax
import jax.numpy as jnp
from jax.experimental import pallas as pl


def kernel(x, edge_index, W1, b1, W2, b2):
    raise NotImplementedError("write your pallas kernel here")


# speedup vs baseline: 7.3439x; 7.3439x over previous
import functools

import jax
import jax.numpy as jnp
from jax import lax
from jax.experimental import pallas as pl
from jax.experimental.pallas import tpu as pltpu
from jax.experimental.pallas import tpu_sc as plsc

N_NODES = 10000
NP = 10240
E = 320000
D_IN = 128
D_HID = 128
D_OUT = 64
C = 128
NW = 32
K = 80
E_PAD = NW * K * C
PER_TILE = K * C
RPS = NP // 16


def _sc_mesh():
    return plsc.VectorSubcoreMesh(core_axis_name="c", subcore_axis_name="s")


def _deg_call():
    @functools.partial(
        pl.kernel,
        out_type=jax.ShapeDtypeStruct((2, NP, D_HID), jnp.float32),
        mesh=_sc_mesh(),
        scratch_types=[
            pltpu.VMEM((K, C), jnp.int32),
            pltpu.VMEM((C, D_HID), jnp.float32),
            pltpu.VMEM_SHARED((NP, D_HID), jnp.float32),
        ],
    )
    def deg_k(dstp_hbm, ones_hbm, zin_hbm, out_hbm, dst_v, ones_v, deg_sh):
        c = lax.axis_index("c")
        s = lax.axis_index("s")
        w = c * 16 + s
        r0 = pl.multiple_of(s * RPS, 8)
        pltpu.sync_copy(zin_hbm, deg_sh.at[pl.ds(r0, RPS)])
        pltpu.sync_copy(ones_hbm, ones_v)
        pltpu.sync_copy(dstp_hbm.at[w], dst_v)
        plsc.subcore_barrier()

        def body(j, carry):
            pltpu.sync_copy(ones_v, deg_sh.at[dst_v.at[j]], add=True)
            return carry

        lax.fori_loop(0, K, body, 0)

        plsc.subcore_barrier()
        pltpu.sync_copy(deg_sh.at[pl.ds(r0, RPS)],
                        out_hbm.at[c].at[pl.ds(r0, RPS)])

    return deg_k


def _agg_call(d):
    @functools.partial(
        pl.kernel,
        out_type=jax.ShapeDtypeStruct((2, NP, d), jnp.float32),
        mesh=_sc_mesh(),
        scratch_types=[
            pltpu.VMEM((K, C), jnp.int32),
            pltpu.VMEM((K, C), jnp.int32),
            pltpu.VMEM((C, d), jnp.float32),
            pltpu.VMEM_SHARED((NP, d), jnp.float32),
            pltpu.SemaphoreType.DMA,
        ],
    )
    def agg_k(g_hbm, srcp_hbm, dstp_hbm, zin_hbm, out_hbm,
              src_v, dst_v, rows_v, acc_sh, sem):
        c = lax.axis_index("c")
        s = lax.axis_index("s")
        w = c * 16 + s
        r0 = pl.multiple_of(s * RPS, 8)
        pltpu.sync_copy(zin_hbm, acc_sh.at[pl.ds(r0, RPS)])
        pltpu.sync_copy(srcp_hbm.at[w], src_v)
        pltpu.sync_copy(dstp_hbm.at[w], dst_v)
        plsc.subcore_barrier()

        def body(j, carry):
            pltpu.async_copy(g_hbm.at[src_v.at[j]], rows_v, sem).wait()
            pltpu.sync_copy(rows_v, acc_sh.at[dst_v.at[j]], add=True)
            return carry

        lax.fori_loop(0, K, body, 0)

        plsc.subcore_barrier()
        pltpu.sync_copy(acc_sh.at[pl.ds(r0, RPS)],
                        out_hbm.at[c].at[pl.ds(r0, RPS)])

    return agg_k


def _dinv(deg_ref):
    total = deg_ref[0] + deg_ref[1] + 1.0
    return lax.rsqrt(total)


def _tc1_call(x_p, w1, deg_parts):
    def body(x_ref, w_ref, deg_ref, g_ref):
        dinv = _dinv(deg_ref)
        g_ref[...] = jnp.dot(x_ref[...], w_ref[...],
                             preferred_element_type=jnp.float32) * dinv

    return pl.pallas_call(
        body, out_shape=jax.ShapeDtypeStruct((NP, D_HID), jnp.float32),
    )(x_p, w1, deg_parts)


def _tc2_call(acc1, g1, deg_parts, w2, b1):
    def body(acc_ref, g1_ref, deg_ref, w_ref, b_ref, g2_ref):
        dinv = _dinv(deg_ref)
        ssum = acc_ref[0] + acc_ref[1] + g1_ref[...]
        h = jnp.maximum(ssum * dinv + b_ref[...], 0.0)
        g2 = jnp.dot(h, w_ref[...], preferred_element_type=jnp.float32) * dinv
        g2_ref[...] = jnp.concatenate(
            [g2, jnp.zeros((NP, D_HID - D_OUT), jnp.float32)], axis=1)

    return pl.pallas_call(
        body, out_shape=jax.ShapeDtypeStruct((NP, D_HID), jnp.float32),
    )(acc1, g1, deg_parts, w2, b1)


def _tc3_call(acc2, g2, deg_parts, b2):
    def body(acc_ref, g2_ref, deg_ref, b_ref, out_ref):
        dinv = _dinv(deg_ref)
        z = (acc_ref[0, :, :D_OUT] + acc_ref[1, :, :D_OUT]
             + g2_ref[:, :D_OUT]) * dinv + b_ref[...]
        z = z - jnp.max(z, axis=1, keepdims=True)
        e = jnp.exp(z)
        out_ref[...] = e / jnp.sum(e, axis=1, keepdims=True)

    return pl.pallas_call(
        body, out_shape=jax.ShapeDtypeStruct((NP, D_OUT), jnp.float32),
    )(acc2, g2, deg_parts, b2)


def kernel(x, edge_index, W1, b1, W2, b2):
    ei = edge_index.astype(jnp.int32)
    src = ei[0]
    dst = ei[1]
    pad = E_PAD - E
    src_p = jnp.concatenate([src, jnp.zeros((pad,), jnp.int32)])
    dst_p = jnp.concatenate(
        [dst, N_NODES + (jnp.arange(pad, dtype=jnp.int32) % (NP - N_NODES))])
    srcp = src_p.reshape(NW, K, C)
    dstp = dst_p.reshape(NW, K, C)
    x_p = jnp.pad(x, ((0, NP - N_NODES), (0, 0)))

    zin = jnp.zeros((RPS, D_HID), jnp.float32)
    deg_raw = _deg_call()(dstp, jnp.ones((C, D_HID), jnp.float32), zin)
    deg_parts = deg_raw[:, :, :1]
    g1 = _tc1_call(x_p, W1, deg_parts)
    acc1 = _agg_call(D_HID)(g1, srcp, dstp, zin)
    g2 = _tc2_call(acc1, g1, deg_parts, W2, b1.reshape(1, -1))
    acc2 = _agg_call(D_HID)(g2, srcp, dstp, zin)
    out = _tc3_call(acc2, g2, deg_parts, b2.reshape(1, -1))
    return out[:N_NODES]


# speedup vs baseline: 7.9749x; 1.0859x over previous
import functools

import jax
import jax.numpy as jnp
from jax import lax
from jax.experimental import pallas as pl
from jax.experimental.pallas import tpu as pltpu
from jax.experimental.pallas import tpu_sc as plsc

N_NODES = 10000
NP = 10240
E = 320000
D_IN = 128
D_HID = 128
D_OUT = 64
C = 128
NW = 32
K = 80
CHUNKS = NW * K
E_PAD = CHUNKS * C
K0 = 40
K1 = 120
RPS = NP // 16


def _sc_mesh():
    return plsc.VectorSubcoreMesh(core_axis_name="c", subcore_axis_name="s")


def _deg_call():
    @functools.partial(
        pl.kernel,
        out_type=jax.ShapeDtypeStruct((2, NP, D_HID), jnp.float32),
        mesh=_sc_mesh(),
        scratch_types=[
            pltpu.VMEM((K, C), jnp.int32),
            pltpu.VMEM((C, D_HID), jnp.float32),
            pltpu.VMEM_SHARED((NP, D_HID), jnp.float32),
        ],
    )
    def deg_k(dstp_hbm, ones_hbm, zin_hbm, out_hbm, dst_v, ones_v, deg_sh):
        c = lax.axis_index("c")
        s = lax.axis_index("s")
        w = c * 16 + s
        r0 = pl.multiple_of(s * RPS, 8)
        pltpu.sync_copy(zin_hbm, deg_sh.at[pl.ds(r0, RPS)])
        pltpu.sync_copy(ones_hbm, ones_v)
        pltpu.sync_copy(dstp_hbm.at[pl.ds(w * K, K)], dst_v)
        plsc.subcore_barrier()

        def body(j, carry):
            pltpu.sync_copy(ones_v, deg_sh.at[dst_v.at[j]], add=True)
            return carry

        lax.fori_loop(0, K, body, 0)

        plsc.subcore_barrier()
        pltpu.sync_copy(deg_sh.at[pl.ds(r0, RPS)],
                        out_hbm.at[c].at[pl.ds(r0, RPS)])

    return deg_k


def _agg_call(d):
    @functools.partial(
        pl.kernel,
        out_type=jax.ShapeDtypeStruct((2, NP, d), jnp.float32),
        mesh=_sc_mesh(),
        scratch_types=[
            pltpu.VMEM((max(K0, K1), C), jnp.int32),
            pltpu.VMEM((max(K0, K1), C), jnp.int32),
            pltpu.VMEM((C, d), jnp.float32),
            pltpu.VMEM_SHARED((NP, d), jnp.float32),
            pltpu.SemaphoreType.DMA,
        ],
    )
    def agg_k(g_hbm, srcp_hbm, dstp_hbm, zin_hbm, out_hbm,
              src_v, dst_v, rows_v, acc_sh, sem):
        c = lax.axis_index("c")
        s = lax.axis_index("s")
        r0 = pl.multiple_of(s * RPS, 8)
        pltpu.sync_copy(zin_hbm, acc_sh.at[pl.ds(r0, RPS)])
        plsc.subcore_barrier()

        def stage_and_run(k, chunk_base):
            pltpu.sync_copy(srcp_hbm.at[pl.ds(chunk_base, k)],
                            src_v.at[pl.ds(0, k)])
            pltpu.sync_copy(dstp_hbm.at[pl.ds(chunk_base, k)],
                            dst_v.at[pl.ds(0, k)])

            def body(j, carry):
                pltpu.async_copy(g_hbm.at[src_v.at[j]], rows_v, sem).wait()
                pltpu.sync_copy(rows_v, acc_sh.at[dst_v.at[j]], add=True)
                return carry

            lax.fori_loop(0, k, body, 0)

        @pl.when(c == 0)
        def _():
            stage_and_run(K0, s * K0)

        @pl.when(c == 1)
        def _():
            stage_and_run(K1, 16 * K0 + s * K1)

        plsc.subcore_barrier()
        pltpu.sync_copy(acc_sh.at[pl.ds(r0, RPS)],
                        out_hbm.at[c].at[pl.ds(r0, RPS)])

    return agg_k


def _dinv(deg_ref):
    total = deg_ref[0] + deg_ref[1] + 1.0
    return lax.rsqrt(total)


def _tc1_call(x_p, w1, deg_parts):
    def body(x_ref, w_ref, deg_ref, g_ref):
        dinv = _dinv(deg_ref)
        g_ref[...] = jnp.dot(x_ref[...], w_ref[...],
                             preferred_element_type=jnp.float32) * dinv

    return pl.pallas_call(
        body, out_shape=jax.ShapeDtypeStruct((NP, D_HID), jnp.float32),
    )(x_p, w1, deg_parts)


def _tc2_call(acc1, g1, deg_parts, w2, b1):
    def body(acc_ref, g1_ref, deg_ref, w_ref, b_ref, g2_ref):
        dinv = _dinv(deg_ref)
        ssum = acc_ref[0] + acc_ref[1] + g1_ref[...]
        h = jnp.maximum(ssum * dinv + b_ref[...], 0.0)
        g2 = jnp.dot(h, w_ref[...], preferred_element_type=jnp.float32) * dinv
        g2_ref[...] = jnp.concatenate(
            [g2, jnp.zeros((NP, D_HID - D_OUT), jnp.float32)], axis=1)

    return pl.pallas_call(
        body, out_shape=jax.ShapeDtypeStruct((NP, D_HID), jnp.float32),
    )(acc1, g1, deg_parts, w2, b1)


def _tc3_call(acc2, g2, deg_parts, b2):
    def body(acc_ref, g2_ref, deg_ref, b_ref, out_ref):
        dinv = _dinv(deg_ref)
        z = (acc_ref[0, :, :D_OUT] + acc_ref[1, :, :D_OUT]
             + g2_ref[:, :D_OUT]) * dinv + b_ref[...]
        z = z - jnp.max(z, axis=1, keepdims=True)
        e = jnp.exp(z)
        out_ref[...] = e / jnp.sum(e, axis=1, keepdims=True)

    return pl.pallas_call(
        body, out_shape=jax.ShapeDtypeStruct((NP, D_OUT), jnp.float32),
    )(acc2, g2, deg_parts, b2)


def kernel(x, edge_index, W1, b1, W2, b2):
    ei = edge_index.astype(jnp.int32)
    src = ei[0]
    dst = ei[1]
    pad = E_PAD - E
    src_p = jnp.concatenate([src, jnp.zeros((pad,), jnp.int32)])
    dst_p = jnp.concatenate(
        [dst, N_NODES + (jnp.arange(pad, dtype=jnp.int32) % (NP - N_NODES))])
    srcp = src_p.reshape(CHUNKS, C)
    dstp = dst_p.reshape(CHUNKS, C)
    x_p = jnp.pad(x, ((0, NP - N_NODES), (0, 0)))

    zin = jnp.zeros((RPS, D_HID), jnp.float32)
    deg_raw = _deg_call()(dstp, jnp.ones((C, D_HID), jnp.float32), zin)
    deg_parts = deg_raw[:, :, :1]
    g1 = _tc1_call(x_p, W1, deg_parts)
    acc1 = _agg_call(D_HID)(g1, srcp, dstp, zin)
    g2 = _tc2_call(acc1, g1, deg_parts, W2, b1.reshape(1, -1))
    acc2 = _agg_call(D_HID)(g2, srcp, dstp, zin)
    out = _tc3_call(acc2, g2, deg_parts, b2.reshape(1, -1))
    return out[:N_NODES]


# speedup vs baseline: 10.3904x; 1.3029x over previous
import functools

import jax
import jax.numpy as jnp
from jax import lax
from jax.experimental import pallas as pl
from jax.experimental.pallas import tpu as pltpu
from jax.experimental.pallas import tpu_sc as plsc

N_NODES = 10000
NP = 10240
E = 320000
D_IN = 128
D_HID = 128
D_OUT = 64
C = 128
NW = 32
K = 80
CHUNKS = NW * K
E_PAD = CHUNKS * C
K0 = 120
K1 = 40
G = 40
RPS = NP // 16


def _sc_mesh():
    return plsc.VectorSubcoreMesh(core_axis_name="c", subcore_axis_name="s")


def _deg_call():
    @functools.partial(
        pl.kernel,
        out_type=jax.ShapeDtypeStruct((2, NP, D_HID), jnp.float32),
        mesh=_sc_mesh(),
        scratch_types=[
            pltpu.VMEM((K, C), jnp.int32),
            pltpu.VMEM((C, D_HID), jnp.float32),
            pltpu.VMEM_SHARED((NP, D_HID), jnp.float32),
            pltpu.SemaphoreType.DMA((8,)),
        ],
    )
    def deg_k(dstp_hbm, ones_hbm, zin_hbm, out_hbm, dst_v, ones_v, deg_sh,
              dsem):
        c = lax.axis_index("c")
        s = lax.axis_index("s")
        w = c * 16 + s
        r0 = pl.multiple_of(s * RPS, 8)
        pltpu.sync_copy(zin_hbm, deg_sh.at[pl.ds(r0, RPS)])
        pltpu.sync_copy(ones_hbm, ones_v)
        pltpu.sync_copy(dstp_hbm.at[pl.ds(w * K, K)], dst_v)
        plsc.subcore_barrier()

        def body(i, carry):
            for b in range(8):
                pltpu.async_copy(ones_v, deg_sh.at[dst_v.at[i * 8 + b]],
                                 dsem.at[b], add=True)
            for b in range(8):
                pltpu.make_async_copy(ones_v, deg_sh.at[dst_v.at[0]],
                                      dsem.at[b]).wait()
            return carry

        lax.fori_loop(0, K // 8, body, 0)

        plsc.subcore_barrier()
        pltpu.sync_copy(deg_sh.at[pl.ds(r0, RPS)],
                        out_hbm.at[c].at[pl.ds(r0, RPS)])

    return deg_k


def _agg_call(d):
    @functools.partial(
        pl.kernel,
        out_type=jax.ShapeDtypeStruct((2, NP, d), jnp.float32),
        mesh=_sc_mesh(),
        scratch_types=[
            pltpu.VMEM((G, C), jnp.int32),
            pltpu.VMEM((G, C), jnp.int32),
            pltpu.VMEM((2, C, d), jnp.float32),
            pltpu.VMEM_SHARED((NP, d), jnp.float32),
            pltpu.SemaphoreType.DMA((2,)),
            pltpu.SemaphoreType.DMA((2,)),
        ],
    )
    def agg_k(g_hbm, srcp_hbm, dstp_hbm, zin_hbm, out_hbm,
              src_v, dst_v, rows_v, acc_sh, gsem, ssem):
        c = lax.axis_index("c")
        s = lax.axis_index("s")
        r0 = pl.multiple_of(s * RPS, 8)
        pltpu.sync_copy(zin_hbm, acc_sh.at[pl.ds(r0, RPS)])
        plsc.subcore_barrier()

        def gather(j, b):
            pltpu.async_copy(g_hbm.at[src_v.at[j]], rows_v.at[b], gsem.at[b])

        def wait_gather(b):
            pltpu.make_async_copy(g_hbm.at[src_v.at[0]], rows_v.at[b],
                                  gsem.at[b]).wait()

        def scatter(j, b):
            pltpu.async_copy(rows_v.at[b], acc_sh.at[dst_v.at[j]],
                             ssem.at[b], add=True)

        def wait_scatter(b):
            pltpu.make_async_copy(rows_v.at[b], acc_sh.at[dst_v.at[0]],
                                  ssem.at[b]).wait()

        def stage_and_run(k, chunk_base):
            def blk(t, carry):
                base = chunk_base + t * G
                pltpu.sync_copy(srcp_hbm.at[pl.ds(base, G)], src_v)
                pltpu.sync_copy(dstp_hbm.at[pl.ds(base, G)], dst_v)
                gather(0, 0)
                gather(1, 1)

                def inner(i, icarry):
                    j0 = 2 * i
                    for b in range(2):
                        wait_gather(b)
                        scatter(j0 + b, b)
                    for b in range(2):
                        wait_scatter(b)

                        @pl.when(j0 + 2 + b < G)
                        def _():
                            gather(j0 + 2 + b, b)
                    return icarry

                lax.fori_loop(0, G // 2, inner, 0)
                return carry

            lax.fori_loop(0, k // G, blk, 0)

        @pl.when(c == 0)
        def _():
            stage_and_run(K0, s * K0)

        @pl.when(c == 1)
        def _():
            stage_and_run(K1, 16 * K0 + s * K1)

        plsc.subcore_barrier()
        pltpu.sync_copy(acc_sh.at[pl.ds(r0, RPS)],
                        out_hbm.at[c].at[pl.ds(r0, RPS)])

    return agg_k


def _dinv(deg_ref):
    total = deg_ref[0] + deg_ref[1] + 1.0
    return lax.rsqrt(total)


def _tc1_call(x_p, w1, deg_parts):
    def body(x_ref, w_ref, deg_ref, g_ref):
        dinv = _dinv(deg_ref)
        g_ref[...] = jnp.dot(x_ref[...], w_ref[...],
                             preferred_element_type=jnp.float32) * dinv

    return pl.pallas_call(
        body, out_shape=jax.ShapeDtypeStruct((NP, D_HID), jnp.float32),
    )(x_p, w1, deg_parts)


def _tc2_call(acc1, g1, deg_parts, w2, b1):
    def body(acc_ref, g1_ref, deg_ref, w_ref, b_ref, g2_ref):
        dinv = _dinv(deg_ref)
        ssum = acc_ref[0] + acc_ref[1] + g1_ref[...]
        h = jnp.maximum(ssum * dinv + b_ref[...], 0.0)
        g2 = jnp.dot(h, w_ref[...], preferred_element_type=jnp.float32) * dinv
        g2_ref[...] = jnp.concatenate(
            [g2, jnp.zeros((NP, D_HID - D_OUT), jnp.float32)], axis=1)

    return pl.pallas_call(
        body, out_shape=jax.ShapeDtypeStruct((NP, D_HID), jnp.float32),
    )(acc1, g1, deg_parts, w2, b1)


def _tc3_call(acc2, g2, deg_parts, b2):
    def body(acc_ref, g2_ref, deg_ref, b_ref, out_ref):
        dinv = _dinv(deg_ref)
        z = (acc_ref[0, :, :D_OUT] + acc_ref[1, :, :D_OUT]
             + g2_ref[:, :D_OUT]) * dinv + b_ref[...]
        z = z - jnp.max(z, axis=1, keepdims=True)
        e = jnp.exp(z)
        out_ref[...] = e / jnp.sum(e, axis=1, keepdims=True)

    return pl.pallas_call(
        body, out_shape=jax.ShapeDtypeStruct((NP, D_OUT), jnp.float32),
    )(acc2, g2, deg_parts, b2)


def kernel(x, edge_index, W1, b1, W2, b2):
    ei = edge_index.astype(jnp.int32)
    src = ei[0]
    dst = ei[1]
    pad = E_PAD - E
    src_p = jnp.concatenate([src, jnp.zeros((pad,), jnp.int32)])
    dst_p = jnp.concatenate(
        [dst, N_NODES + (jnp.arange(pad, dtype=jnp.int32) % (NP - N_NODES))])
    srcp = src_p.reshape(CHUNKS, C)
    dstp = dst_p.reshape(CHUNKS, C)
    x_p = jnp.pad(x, ((0, NP - N_NODES), (0, 0)))

    zin = jnp.zeros((RPS, D_HID), jnp.float32)
    deg_raw = _deg_call()(dstp, jnp.ones((C, D_HID), jnp.float32), zin)
    deg_parts = deg_raw[:, :, :1]
    g1 = _tc1_call(x_p, W1, deg_parts)
    acc1 = _agg_call(D_HID)(g1, srcp, dstp, zin)
    g2 = _tc2_call(acc1, g1, deg_parts, W2, b1.reshape(1, -1))
    acc2 = _agg_call(D_HID)(g2, srcp, dstp, zin)
    out = _tc3_call(acc2, g2, deg_parts, b2.reshape(1, -1))
    return out[:N_NODES]


# speedup vs baseline: 10.4578x; 1.0065x over previous
import functools

import jax
import jax.numpy as jnp
from jax import lax
from jax.experimental import pallas as pl
from jax.experimental.pallas import tpu as pltpu
from jax.experimental.pallas import tpu_sc as plsc

N_NODES = 10000
NP = 10240
E = 320000
D_IN = 128
D_HID = 128
D_OUT = 64
C = 128
NW = 32
K = 80
CHUNKS = NW * K
E_PAD = CHUNKS * C
K0 = 128
K1 = 32
G0 = 32
G1 = 32
GMAX = max(G0, G1)
RPS = NP // 16


def _sc_mesh():
    return plsc.VectorSubcoreMesh(core_axis_name="c", subcore_axis_name="s")


def _deg_call():
    @functools.partial(
        pl.kernel,
        out_type=jax.ShapeDtypeStruct((2, NP, D_HID), jnp.float32),
        mesh=_sc_mesh(),
        scratch_types=[
            pltpu.VMEM((K, C), jnp.int32),
            pltpu.VMEM((C, D_HID), jnp.float32),
            pltpu.VMEM_SHARED((NP, D_HID), jnp.float32),
            pltpu.SemaphoreType.DMA((8,)),
        ],
    )
    def deg_k(dstp_hbm, ones_hbm, zin_hbm, out_hbm, dst_v, ones_v, deg_sh,
              dsem):
        c = lax.axis_index("c")
        s = lax.axis_index("s")
        w = c * 16 + s
        r0 = pl.multiple_of(s * RPS, 8)
        pltpu.sync_copy(zin_hbm, deg_sh.at[pl.ds(r0, RPS)])
        pltpu.sync_copy(ones_hbm, ones_v)
        pltpu.sync_copy(dstp_hbm.at[pl.ds(w * K, K)], dst_v)
        plsc.subcore_barrier()

        def body(i, carry):
            for b in range(8):
                pltpu.async_copy(ones_v, deg_sh.at[dst_v.at[i * 8 + b]],
                                 dsem.at[b], add=True)
            for b in range(8):
                pltpu.make_async_copy(ones_v, deg_sh.at[dst_v.at[0]],
                                      dsem.at[b]).wait()
            return carry

        lax.fori_loop(0, K // 8, body, 0)

        plsc.subcore_barrier()
        pltpu.sync_copy(deg_sh.at[pl.ds(r0, RPS)],
                        out_hbm.at[c].at[pl.ds(r0, RPS)])

    return deg_k


def _agg_call(d):
    @functools.partial(
        pl.kernel,
        out_type=jax.ShapeDtypeStruct((2, NP, d), jnp.float32),
        mesh=_sc_mesh(),
        scratch_types=[
            pltpu.VMEM((GMAX, C), jnp.int32),
            pltpu.VMEM((GMAX, C), jnp.int32),
            pltpu.VMEM((2, C, d), jnp.float32),
            pltpu.VMEM_SHARED((NP, d), jnp.float32),
            pltpu.SemaphoreType.DMA((2,)),
            pltpu.SemaphoreType.DMA((2,)),
        ],
    )
    def agg_k(g_hbm, srcp_hbm, dstp_hbm, zin_hbm, out_hbm,
              src_v, dst_v, rows_v, acc_sh, gsem, ssem):
        c = lax.axis_index("c")
        s = lax.axis_index("s")
        r0 = pl.multiple_of(s * RPS, 8)
        pltpu.sync_copy(zin_hbm, acc_sh.at[pl.ds(r0, RPS)])
        plsc.subcore_barrier()

        def gather(j, b):
            pltpu.async_copy(g_hbm.at[src_v.at[j]], rows_v.at[b], gsem.at[b])

        def wait_gather(b):
            pltpu.make_async_copy(g_hbm.at[src_v.at[0]], rows_v.at[b],
                                  gsem.at[b]).wait()

        def scatter(j, b):
            pltpu.async_copy(rows_v.at[b], acc_sh.at[dst_v.at[j]],
                             ssem.at[b], add=True)

        def wait_scatter(b):
            pltpu.make_async_copy(rows_v.at[b], acc_sh.at[dst_v.at[0]],
                                  ssem.at[b]).wait()

        def stage_and_run(k, chunk_base, g):
            def blk(t, carry):
                base = pl.multiple_of(chunk_base + t * g, 8)
                pltpu.sync_copy(srcp_hbm.at[pl.ds(base, g)],
                                src_v.at[pl.ds(0, g)])
                pltpu.sync_copy(dstp_hbm.at[pl.ds(base, g)],
                                dst_v.at[pl.ds(0, g)])
                gather(0, 0)
                gather(1, 1)

                def inner(i, icarry):
                    j0 = 2 * i
                    for b in range(2):
                        wait_gather(b)
                        scatter(j0 + b, b)
                    for b in range(2):
                        wait_scatter(b)

                        @pl.when(j0 + 2 + b < g)
                        def _():
                            gather(j0 + 2 + b, b)
                    return icarry

                lax.fori_loop(0, g // 2, inner, 0)
                return carry

            lax.fori_loop(0, k // g, blk, 0)

        @pl.when(c == 0)
        def _():
            stage_and_run(K0, s * K0, G0)

        @pl.when(c == 1)
        def _():
            stage_and_run(K1, 16 * K0 + s * K1, G1)

        plsc.subcore_barrier()
        pltpu.sync_copy(acc_sh.at[pl.ds(r0, RPS)],
                        out_hbm.at[c].at[pl.ds(r0, RPS)])

    return agg_k


def _dinv(deg_ref):
    total = deg_ref[0] + deg_ref[1] + 1.0
    return lax.rsqrt(total)


def _tc1_call(x_p, w1, deg_parts):
    def body(x_ref, w_ref, deg_ref, g_ref):
        dinv = _dinv(deg_ref)
        g_ref[...] = jnp.dot(x_ref[...], w_ref[...],
                             preferred_element_type=jnp.float32) * dinv

    return pl.pallas_call(
        body, out_shape=jax.ShapeDtypeStruct((NP, D_HID), jnp.float32),
    )(x_p, w1, deg_parts)


def _tc2_call(acc1, g1, deg_parts, w2, b1):
    def body(acc_ref, g1_ref, deg_ref, w_ref, b_ref, g2_ref):
        dinv = _dinv(deg_ref)
        ssum = acc_ref[0] + acc_ref[1] + g1_ref[...]
        h = jnp.maximum(ssum * dinv + b_ref[...], 0.0)
        g2 = jnp.dot(h, w_ref[...], preferred_element_type=jnp.float32) * dinv
        g2_ref[...] = jnp.concatenate(
            [g2, jnp.zeros((NP, D_HID - D_OUT), jnp.float32)], axis=1)

    return pl.pallas_call(
        body, out_shape=jax.ShapeDtypeStruct((NP, D_HID), jnp.float32),
    )(acc1, g1, deg_parts, w2, b1)


def _tc3_call(acc2, g2, deg_parts, b2):
    def body(acc_ref, g2_ref, deg_ref, b_ref, out_ref):
        dinv = _dinv(deg_ref)
        z = (acc_ref[0, :, :D_OUT] + acc_ref[1, :, :D_OUT]
             + g2_ref[:, :D_OUT]) * dinv + b_ref[...]
        z = z - jnp.max(z, axis=1, keepdims=True)
        e = jnp.exp(z)
        out_ref[...] = e / jnp.sum(e, axis=1, keepdims=True)

    return pl.pallas_call(
        body, out_shape=jax.ShapeDtypeStruct((NP, D_OUT), jnp.float32),
    )(acc2, g2, deg_parts, b2)


def kernel(x, edge_index, W1, b1, W2, b2):
    ei = edge_index.astype(jnp.int32)
    src = ei[0]
    dst = ei[1]
    pad = E_PAD - E
    src_p = jnp.concatenate([src, jnp.zeros((pad,), jnp.int32)])
    dst_p = jnp.concatenate(
        [dst, N_NODES + (jnp.arange(pad, dtype=jnp.int32) % (NP - N_NODES))])
    srcp = src_p.reshape(CHUNKS, C)
    dstp = dst_p.reshape(CHUNKS, C)
    x_p = jnp.pad(x, ((0, NP - N_NODES), (0, 0)))

    zin = jnp.zeros((RPS, D_HID), jnp.float32)
    deg_raw = _deg_call()(dstp, jnp.ones((C, D_HID), jnp.float32), zin)
    deg_parts = deg_raw[:, :, :1]
    g1 = _tc1_call(x_p, W1, deg_parts)
    acc1 = _agg_call(D_HID)(g1, srcp, dstp, zin)
    g2 = _tc2_call(acc1, g1, deg_parts, W2, b1.reshape(1, -1))
    acc2 = _agg_call(D_HID)(g2, srcp, dstp, zin)
    out = _tc3_call(acc2, g2, deg_parts, b2.reshape(1, -1))
    return out[:N_NODES]
